# Initial kernel scaffold; baseline (speedup 1.0000x reference)
import jax
import jax.numpy as jnp
from jax import lax
from jax.experimental import pallas as pl
from jax.experimental.pallas import tpu as pltpu
from jax.experimental.pallas import tpu_sc as plsc

N = 50000
D = 256
B = 128
NC = 2
NS = 16
NW = NC * NS
T = 128
SEG = 2 * B
ACC_ROWS = 272
N8 = N // 8
MAX_LEN = 8 * ((N8 + NW - 1) // NW + 1)
N_TILES = (MAX_LEN + T - 1) // T


def _worker_base(w):
    return 8 * ((w * N8) // NW)


def _sc_body(out_hbm, zv_hbm, bat_hbm, zeros_hbm, parts_hbm,
             rowbuf, zvbuf, batbuf, idxbuf, zbuf, drain, acc):
    c = lax.axis_index("c")
    s = lax.axis_index("s")
    w = c * NS + s

    pltpu.sync_copy(zeros_hbm, zbuf)
    pltpu.sync_copy(zbuf, acc.at[pl.ds(s * 17, 17)])
    plsc.subcore_barrier()

    base = _worker_base(w)
    wend = _worker_base(w + 1)
    lane = lax.iota(jnp.int32, 16)

    for j in range(N_TILES):
        ub = base + j * T
        tb = jnp.minimum(ub, wend - T)
        delta = ub - tb

        pltpu.sync_copy(zv_hbm.at[pl.ds(tb, T)], zvbuf)
        pltpu.sync_copy(bat_hbm.at[pl.ds(tb, T)], batbuf)
        for g in range(T // 16):
            z = zvbuf[pl.ds(g * 16, 16)]
            bt = batbuf[pl.ds(g * 16, 16)]
            idx = bt * 2 + (z == 100).astype(jnp.int32)
            t = lane + (g * 16)
            idx = jnp.where(t >= delta, idx, SEG)
            idxbuf[j, pl.ds(g * 16, 16)] = idx

        pltpu.sync_copy(out_hbm.at[pl.ds(tb, T)], rowbuf)
        pltpu.sync_copy(rowbuf, acc.at[idxbuf.at[j]], add=True)

    plsc.subcore_barrier()
    pltpu.sync_copy(acc.at[pl.ds(s * 16, 16)], drain)
    pltpu.sync_copy(drain, parts_hbm.at[c, pl.ds(s * 16, 16)])


_sc_pool = pl.kernel(
    _sc_body,
    out_type=jax.ShapeDtypeStruct((NC, SEG, D), jnp.float32),
    mesh=plsc.VectorSubcoreMesh(core_axis_name="c", subcore_axis_name="s"),
    scratch_types=[
        pltpu.VMEM((T, D), jnp.float32),
        pltpu.VMEM((T,), jnp.int32),
        pltpu.VMEM((T,), jnp.int32),
        pltpu.VMEM((N_TILES, T), jnp.int32),
        pltpu.VMEM((17, D), jnp.float32),
        pltpu.VMEM((16, D), jnp.float32),
        pltpu.VMEM_SHARED((ACC_ROWS, D), jnp.float32),
    ],
)


def _combine_body(p_ref, o_ref):
    o_ref[...] = p_ref[0] + p_ref[1]


_combine = pl.pallas_call(
    _combine_body,
    out_shape=jax.ShapeDtypeStruct((SEG, D), jnp.float32),
)


def kernel(out, zv, x_rv_batch):
    zv32 = zv.astype(jnp.int32)
    bat32 = x_rv_batch.astype(jnp.int32)
    zeros = jnp.zeros((17, D), jnp.float32)
    parts = _sc_pool(out, zv32, bat32, zeros)
    summed = _combine(parts)
    return summed.reshape(B, 2 * D)


# speedup vs baseline: 4.9933x; 4.9933x over previous
import jax
import jax.numpy as jnp
from jax import lax
from jax.experimental import pallas as pl
from jax.experimental.pallas import tpu as pltpu
from jax.experimental.pallas import tpu_sc as plsc

N = 50000
D = 256
B = 128
NC = 2
NS = 16
NW = NC * NS
T = 128
SEG = 2 * B
ACC_ROWS = SEG + 16
N8 = N // 8
MAX_LEN = 8 * ((N8 + NW - 1) // NW + 1)
N_TILES = (MAX_LEN + T - 1) // T


def _worker_base(w):
    return 8 * ((w * N8) // NW)


def _sc_body(out_hbm, zv_hbm, bat_hbm, zeros_hbm, parts_hbm,
             rowbuf, zvbuf, batbuf, acc):
    c = lax.axis_index("c")
    s = lax.axis_index("s")
    w = c * NS + s

    pltpu.sync_copy(zeros_hbm, acc)

    base = _worker_base(w)
    wend = _worker_base(w + 1)
    lane = lax.iota(jnp.int32, 16)

    def tile_body(j, carry):
        ub = base + j * T
        tb = jnp.minimum(ub, wend - T)
        delta = ub - tb

        pltpu.sync_copy(zv_hbm.at[pl.ds(tb, T)], zvbuf)
        pltpu.sync_copy(bat_hbm.at[pl.ds(tb, T)], batbuf)
        pltpu.sync_copy(out_hbm.at[pl.ds(tb, T)], rowbuf)

        def grp_body(g, carry2):
            z = zvbuf[pl.ds(g * 16, 16)]
            bt = batbuf[pl.ds(g * 16, 16)]
            segv = bt * 2 + jnp.where(z == 100, 1, 0)
            valid = (lane + g * 16) >= delta
            segv = jnp.where(valid, segv, SEG)
            for r in range(16):
                seg16 = jnp.full((16,), segv[r], jnp.int32)
                for jj in range(D // 16):
                    val = rowbuf[g * 16 + r, pl.ds(jj * 16, 16)]
                    plsc.addupdate_scatter(
                        acc, [seg16, jj * 16 + lane], val)
            return carry2

        lax.fori_loop(0, T // 16, grp_body, 0)
        return carry

    lax.fori_loop(0, N_TILES, tile_body, 0)

    pltpu.sync_copy(acc.at[pl.ds(0, SEG)], parts_hbm.at[w])


_sc_pool = pl.kernel(
    _sc_body,
    out_type=jax.ShapeDtypeStruct((NW, SEG, D), jnp.float32),
    mesh=plsc.VectorSubcoreMesh(core_axis_name="c", subcore_axis_name="s"),
    compiler_params=pltpu.CompilerParams(needs_layout_passes=False),
    scratch_types=[
        pltpu.VMEM((T, D), jnp.float32),
        pltpu.VMEM((T,), jnp.int32),
        pltpu.VMEM((T,), jnp.int32),
        pltpu.VMEM((ACC_ROWS, D), jnp.float32),
    ],
)


def _combine_body(p_ref, o_ref):
    acc = p_ref[0]
    for i in range(1, NW):
        acc = acc + p_ref[i]
    o_ref[...] = acc


_combine = pl.pallas_call(
    _combine_body,
    out_shape=jax.ShapeDtypeStruct((SEG, D), jnp.float32),
)


def kernel(out, zv, x_rv_batch):
    zv32 = zv.astype(jnp.int32)
    bat32 = x_rv_batch.astype(jnp.int32)
    zeros = jnp.zeros((ACC_ROWS, D), jnp.float32)
    parts = _sc_pool(out, zv32, bat32, zeros)
    summed = _combine(parts)
    return summed.reshape(B, 2 * D)


# speedup vs baseline: 6.3014x; 1.2620x over previous
import jax
import jax.numpy as jnp
from jax import lax
from jax.experimental import pallas as pl
from jax.experimental.pallas import tpu as pltpu
from jax.experimental.pallas import tpu_sc as plsc

N = 50000
D = 256
B = 128
NC = 2
NS = 16
NW = NC * NS
T = 96
SEG = 2 * B
ACC_ROWS = SEG + 16
N8 = N // 8
CHUNK = 8 * ((N8 + NW - 1) // NW + 1)
N_TILES = (CHUNK + T - 1) // T
N_PAIRS = N_TILES // 2


def _worker_base(w):
    return 8 * ((w * N8) // NW)


def _sc_body(out_hbm, zv_hbm, bat_hbm, zeros_hbm, parts_hbm,
             rowa, rowb, zvbuf, batbuf, acc, sema, semb):
    c = lax.axis_index("c")
    s = lax.axis_index("s")
    w = c * NS + s

    base = _worker_base(w)
    wend = _worker_base(w + 1)
    cb = jnp.minimum(base, N - CHUNK)
    lane = lax.iota(jnp.int32, 16)

    def tile_base(j):
        return jnp.minimum(base + j * T, wend - T)

    pltpu.async_copy(out_hbm.at[pl.ds(tile_base(0), T)], rowa, sema)
    pltpu.async_copy(out_hbm.at[pl.ds(tile_base(1), T)], rowb, semb)
    pltpu.sync_copy(zv_hbm.at[pl.ds(cb, CHUNK)], zvbuf)
    pltpu.sync_copy(bat_hbm.at[pl.ds(cb, CHUNK)], batbuf)
    pltpu.sync_copy(zeros_hbm, acc)

    def process(buf, j):
        tb = tile_base(j)
        delta = base + j * T - tb
        rel = tb - cb

        def grp_body(g, carry):
            z = zvbuf[pl.ds(rel + g * 16, 16)]
            bt = batbuf[pl.ds(rel + g * 16, 16)]
            segv = bt * 2 + jnp.where(z == 100, 1, 0)
            valid = (lane + g * 16) >= delta
            segv = jnp.where(valid, segv, SEG)
            for r in range(16):
                seg16 = jnp.full((16,), segv[r], jnp.int32)
                for jj in range(D // 16):
                    val = buf[g * 16 + r, pl.ds(jj * 16, 16)]
                    plsc.addupdate_scatter(
                        acc, [seg16, jj * 16 + lane], val)
            return carry

        lax.fori_loop(0, T // 16, grp_body, 0)

    def pair_body(p, carry):
        j0 = 2 * p
        pltpu.make_async_copy(out_hbm.at[pl.ds(0, T)], rowa, sema).wait()
        process(rowa, j0)

        @pl.when(j0 + 2 < N_TILES)
        def _next_a():
            pltpu.async_copy(out_hbm.at[pl.ds(tile_base(j0 + 2), T)],
                             rowa, sema)

        pltpu.make_async_copy(out_hbm.at[pl.ds(0, T)], rowb, semb).wait()
        process(rowb, j0 + 1)

        @pl.when(j0 + 3 < N_TILES)
        def _next_b():
            pltpu.async_copy(out_hbm.at[pl.ds(tile_base(j0 + 3), T)],
                             rowb, semb)

        return carry

    lax.fori_loop(0, N_PAIRS, pair_body, 0)

    pltpu.make_async_copy(out_hbm.at[pl.ds(0, T)], rowa, sema).wait()
    process(rowa, N_TILES - 1)

    pltpu.sync_copy(acc.at[pl.ds(0, SEG)], parts_hbm.at[w])


_sc_pool = pl.kernel(
    _sc_body,
    out_type=jax.ShapeDtypeStruct((NW, SEG, D), jnp.float32),
    mesh=plsc.VectorSubcoreMesh(core_axis_name="c", subcore_axis_name="s"),
    compiler_params=pltpu.CompilerParams(needs_layout_passes=False),
    scratch_types=[
        pltpu.VMEM((T, D), jnp.float32),
        pltpu.VMEM((T, D), jnp.float32),
        pltpu.VMEM((CHUNK,), jnp.int32),
        pltpu.VMEM((CHUNK,), jnp.int32),
        pltpu.VMEM((ACC_ROWS, D), jnp.float32),
        pltpu.SemaphoreType.DMA,
        pltpu.SemaphoreType.DMA,
    ],
)


def _combine_body(p_ref, o_ref):
    acc = p_ref[0]
    for i in range(1, NW):
        acc = acc + p_ref[i]
    o_ref[...] = acc


_combine = pl.pallas_call(
    _combine_body,
    out_shape=jax.ShapeDtypeStruct((SEG, D), jnp.float32),
)


def kernel(out, zv, x_rv_batch):
    zv32 = zv.astype(jnp.int32)
    bat32 = x_rv_batch.astype(jnp.int32)
    zeros = jnp.zeros((ACC_ROWS, D), jnp.float32)
    parts = _sc_pool(out, zv32, bat32, zeros)
    summed = _combine(parts)
    return summed.reshape(B, 2 * D)


# speedup vs baseline: 10.7317x; 1.7031x over previous
import jax
import jax.numpy as jnp
from jax import lax
from jax.experimental import pallas as pl
from jax.experimental.pallas import tpu as pltpu
from jax.experimental.pallas import tpu_sc as plsc

N = 50000
D = 256
B = 128
NC = 2
NS = 16
NW = NC * NS
T = 96
G = 16
SEG = 2 * B
ACC_ROWS = SEG + 16
N8 = N // 8
CHUNK = 8 * ((N8 + NW - 1) // NW + 1)
N_TILES = (CHUNK + T - 1) // T + 1
N_PAIRS = N_TILES // 2


def _worker_base(w):
    return 8 * ((w * N8) // NW)


def _sc_body(out_hbm, zv_hbm, bat_hbm, zeros_hbm, parts_hbm,
             rowa, rowb, zvbuf, batbuf, acc, sema, semb):
    c = lax.axis_index("c")
    s = lax.axis_index("s")
    w = c * NS + s

    base = _worker_base(w)
    wend = _worker_base(w + 1)
    cb = jnp.minimum(base, N - CHUNK)
    lane = lax.iota(jnp.int32, 16)
    zero16 = jnp.zeros((16,), jnp.float32)

    def tile_base(j):
        return jnp.minimum(base + j * T, wend - T)

    pltpu.async_copy(out_hbm.at[pl.ds(tile_base(0), T)], rowa, sema)
    pltpu.async_copy(out_hbm.at[pl.ds(tile_base(1), T)], rowb, semb)
    pltpu.sync_copy(zv_hbm.at[pl.ds(cb, CHUNK)], zvbuf)
    pltpu.sync_copy(bat_hbm.at[pl.ds(cb, CHUNK)], batbuf)
    pltpu.sync_copy(zeros_hbm, acc)

    def flush(carry):
        cur_b, srs, sas = carry
        off_r = 2 * jnp.maximum(cur_b, 0) * D
        for jj in range(D // 16):
            o1 = off_r + jj * 16
            acc[pl.ds(o1, 16)] = acc[pl.ds(o1, 16)] + srs[jj]
            o2 = off_r + D + jj * 16
            acc[pl.ds(o2, 16)] = acc[pl.ds(o2, 16)] + (sas[jj] - srs[jj])
        zeros = tuple(zero16 for _ in range(D // 16))
        return zeros, zeros

    def process(buf, j, carry):
        tb = tile_base(j)
        delta = base + j * T - tb
        rel = tb - cb

        def grp_body(g, carry):
            z = zvbuf[pl.ds(rel + g * G, 16)]
            bt = batbuf[pl.ds(rel + g * G, 16)]
            fast = (bt[0] == bt[15]) & ((g * G) >= delta)

            def fast_fn(carry):
                cur_b, srs, sas = carry
                b = bt[0]

                def keep(carry):
                    return carry[1], carry[2]

                srs, sas = lax.cond(b != cur_b, flush, keep,
                                    (cur_b, srs, sas))
                srl, sal = list(srs), list(sas)
                for r in range(G):
                    mreal = jnp.full((16,), z[r], jnp.int32) != 100
                    for jj in range(D // 16):
                        v = buf[g * G + r, pl.ds(jj * 16, 16)]
                        sal[jj] = sal[jj] + v
                        srl[jj] = srl[jj] + jnp.where(mreal, v, 0.0)
                return b, tuple(srl), tuple(sal)

            def slow_fn(carry):
                segv = bt * 2 + jnp.where(z == 100, 1, 0)
                valid = (lane + g * G) >= delta
                segv = jnp.where(valid, segv * D, SEG * D)
                for r in range(G):
                    seg16 = jnp.full((16,), segv[r], jnp.int32)
                    for jj in range(D // 16):
                        val = buf[g * G + r, pl.ds(jj * 16, 16)]
                        plsc.addupdate_scatter(
                            acc, [seg16 + (jj * 16) + lane], val)
                return carry

            return lax.cond(fast, fast_fn, slow_fn, carry)

        return lax.fori_loop(0, T // G, grp_body, carry)

    def pair_body(p, carry):
        j0 = 2 * p
        pltpu.make_async_copy(out_hbm.at[pl.ds(0, T)], rowa, sema).wait()
        carry = process(rowa, j0, carry)

        @pl.when(j0 + 2 < N_TILES)
        def _next_a():
            pltpu.async_copy(out_hbm.at[pl.ds(tile_base(j0 + 2), T)],
                             rowa, sema)

        pltpu.make_async_copy(out_hbm.at[pl.ds(0, T)], rowb, semb).wait()
        carry = process(rowb, j0 + 1, carry)

        @pl.when(j0 + 3 < N_TILES)
        def _next_b():
            pltpu.async_copy(out_hbm.at[pl.ds(tile_base(j0 + 3), T)],
                             rowb, semb)

        return carry

    zeros0 = tuple(zero16 for _ in range(D // 16))
    carry = (jnp.int32(-1), zeros0, zeros0)
    carry = lax.fori_loop(0, N_PAIRS, pair_body, carry)
    flush(carry)

    pltpu.sync_copy(acc.at[pl.ds(0, SEG * D)], parts_hbm.at[w])


_sc_pool = pl.kernel(
    _sc_body,
    out_type=jax.ShapeDtypeStruct((NW, SEG * D), jnp.float32),
    mesh=plsc.VectorSubcoreMesh(core_axis_name="c", subcore_axis_name="s"),
    compiler_params=pltpu.CompilerParams(needs_layout_passes=False),
    scratch_types=[
        pltpu.VMEM((T, D), jnp.float32),
        pltpu.VMEM((T, D), jnp.float32),
        pltpu.VMEM((CHUNK,), jnp.int32),
        pltpu.VMEM((CHUNK,), jnp.int32),
        pltpu.VMEM((ACC_ROWS * D,), jnp.float32),
        pltpu.SemaphoreType.DMA,
        pltpu.SemaphoreType.DMA,
    ],
)


def _combine_body(p_ref, o_ref):
    acc = p_ref[0]
    for i in range(1, NW):
        acc = acc + p_ref[i]
    o_ref[...] = acc


_combine = pl.pallas_call(
    _combine_body,
    out_shape=jax.ShapeDtypeStruct((SEG * D,), jnp.float32),
)


def kernel(out, zv, x_rv_batch):
    zv32 = zv.astype(jnp.int32)
    bat32 = x_rv_batch.astype(jnp.int32)
    zeros = jnp.zeros((ACC_ROWS * D,), jnp.float32)
    parts = _sc_pool(out, zv32, bat32, zeros)
    summed = _combine(parts)
    return summed.reshape(B, 2 * D)


# speedup vs baseline: 12.1645x; 1.1335x over previous
import jax
import jax.numpy as jnp
from jax import lax
from jax.experimental import pallas as pl
from jax.experimental.pallas import tpu as pltpu
from jax.experimental.pallas import tpu_sc as plsc

N = 50000
D = 256
B = 128
NC = 2
NS = 16
NW = NC * NS
T = 96
G = 16
SEG = 2 * B
ACC_ROWS = SEG + 16
NTC = 20000
NSC8 = (N - NTC) // 8
CHUNK = 8 * ((NSC8 + NW - 1) // NW + 1)
_NT_RAW = (CHUNK + T - 1) // T
N_TILES = _NT_RAW + (_NT_RAW % 2)
N_PAIRS = N_TILES // 2
RB = 2000
NB = NTC // RB


def _worker_base(w):
    return NTC + 8 * ((w * NSC8) // NW)


def _sc_body(out_hbm, zv_hbm, bat_hbm, zeros_hbm, parts_hbm,
             rowa, rowb, zvbuf, batbuf, acc, sema, semb):
    c = lax.axis_index("c")
    s = lax.axis_index("s")
    w = c * NS + s

    base = _worker_base(w)
    wend = _worker_base(w + 1)
    cb = jnp.minimum(base, N - CHUNK)
    lane = lax.iota(jnp.int32, 16)
    zero16 = jnp.zeros((16,), jnp.float32)

    def tile_base(j):
        return jnp.minimum(base + j * T, wend - T)

    pltpu.async_copy(out_hbm.at[pl.ds(tile_base(0), T)], rowa, sema)
    pltpu.async_copy(out_hbm.at[pl.ds(tile_base(1), T)], rowb, semb)
    pltpu.sync_copy(zv_hbm.at[pl.ds(cb, CHUNK)], zvbuf)
    pltpu.sync_copy(bat_hbm.at[pl.ds(cb, CHUNK)], batbuf)
    pltpu.sync_copy(zeros_hbm, acc)

    def flush(carry):
        cur_b, srs, sas = carry
        off_r = 2 * jnp.maximum(cur_b, 0) * D
        for jj in range(D // 16):
            o1 = off_r + jj * 16
            acc[pl.ds(o1, 16)] = acc[pl.ds(o1, 16)] + srs[jj]
            o2 = off_r + D + jj * 16
            acc[pl.ds(o2, 16)] = acc[pl.ds(o2, 16)] + (sas[jj] - srs[jj])
        zeros = tuple(zero16 for _ in range(D // 16))
        return zeros, zeros

    def process(buf, j, carry):
        tb = tile_base(j)
        delta = base + j * T - tb
        rel = tb - cb

        def grp_body(g, carry):
            z = zvbuf[pl.ds(rel + g * G, 16)]
            bt = batbuf[pl.ds(rel + g * G, 16)]
            fast = (bt[0] == bt[15]) & ((g * G) >= delta)

            def fast_fn(carry):
                cur_b, srs, sas = carry
                b = bt[0]

                def keep(carry):
                    return carry[1], carry[2]

                srs, sas = lax.cond(b != cur_b, flush, keep,
                                    (cur_b, srs, sas))
                srl, sal = list(srs), list(sas)
                for r in range(G):
                    mreal = jnp.full((16,), z[r], jnp.int32) != 100
                    for jj in range(D // 16):
                        v = buf[g * G + r, pl.ds(jj * 16, 16)]
                        sal[jj] = sal[jj] + v
                        srl[jj] = srl[jj] + jnp.where(mreal, v, 0.0)
                return b, tuple(srl), tuple(sal)

            def slow_fn(carry):
                segv = bt * 2 + jnp.where(z == 100, 1, 0)
                valid = (lane + g * G) >= delta
                segv = jnp.where(valid, segv * D, SEG * D)
                for r in range(G):
                    seg16 = jnp.full((16,), segv[r], jnp.int32)
                    for jj in range(D // 16):
                        val = buf[g * G + r, pl.ds(jj * 16, 16)]
                        plsc.addupdate_scatter(
                            acc, [seg16 + (jj * 16) + lane], val)
                return carry

            return lax.cond(fast, fast_fn, slow_fn, carry)

        return lax.fori_loop(0, T // G, grp_body, carry)

    def pair_body(p, carry):
        j0 = 2 * p
        pltpu.make_async_copy(out_hbm.at[pl.ds(0, T)], rowa, sema).wait()
        carry = process(rowa, j0, carry)

        @pl.when(j0 + 2 < N_TILES)
        def _next_a():
            pltpu.async_copy(out_hbm.at[pl.ds(tile_base(j0 + 2), T)],
                             rowa, sema)

        pltpu.make_async_copy(out_hbm.at[pl.ds(0, T)], rowb, semb).wait()
        carry = process(rowb, j0 + 1, carry)

        @pl.when(j0 + 3 < N_TILES)
        def _next_b():
            pltpu.async_copy(out_hbm.at[pl.ds(tile_base(j0 + 3), T)],
                             rowb, semb)

        return carry

    zeros0 = tuple(zero16 for _ in range(D // 16))
    carry = (jnp.int32(-1), zeros0, zeros0)
    carry = lax.fori_loop(0, N_PAIRS, pair_body, carry)
    flush(carry)

    pltpu.sync_copy(acc.at[pl.ds(0, SEG * D)], parts_hbm.at[w])


_sc_pool = pl.kernel(
    _sc_body,
    out_type=jax.ShapeDtypeStruct((NW, SEG * D), jnp.float32),
    mesh=plsc.VectorSubcoreMesh(core_axis_name="c", subcore_axis_name="s"),
    compiler_params=pltpu.CompilerParams(needs_layout_passes=False),
    scratch_types=[
        pltpu.VMEM((T, D), jnp.float32),
        pltpu.VMEM((T, D), jnp.float32),
        pltpu.VMEM((CHUNK,), jnp.int32),
        pltpu.VMEM((CHUNK,), jnp.int32),
        pltpu.VMEM((ACC_ROWS * D,), jnp.float32),
        pltpu.SemaphoreType.DMA,
        pltpu.SemaphoreType.DMA,
    ],
)


def _tc_body(zv_ref, bat_ref, rows_ref, o_ref):
    i = pl.program_id(0)
    sv = bat_ref[0, 0, :] * 2 + jnp.where(zv_ref[0, 0, :] == 100, 1, 0)
    oh = (lax.broadcasted_iota(jnp.int32, (SEG, RB), 0)
          == sv[None, :]).astype(jnp.bfloat16)
    part = jnp.dot(oh, rows_ref[...].astype(jnp.bfloat16),
                   preferred_element_type=jnp.float32)

    @pl.when(i == 0)
    def _init():
        o_ref[...] = part

    @pl.when(i > 0)
    def _accum():
        o_ref[...] += part


_tc_pool = pl.pallas_call(
    _tc_body,
    grid=(NB,),
    in_specs=[
        pl.BlockSpec((1, 1, RB), lambda i: (i, 0, 0)),
        pl.BlockSpec((1, 1, RB), lambda i: (i, 0, 0)),
        pl.BlockSpec((RB, D), lambda i: (i, 0)),
    ],
    out_specs=pl.BlockSpec((SEG, D), lambda i: (0, 0)),
    out_shape=jax.ShapeDtypeStruct((SEG, D), jnp.float32),
)


def _combine_body(p_ref, t_ref, o_ref):
    acc = t_ref[...]
    for i in range(NW):
        acc = acc + p_ref[i]
    o_ref[...] = acc


_combine = pl.pallas_call(
    _combine_body,
    out_shape=jax.ShapeDtypeStruct((SEG * D,), jnp.float32),
)


def kernel(out, zv, x_rv_batch):
    zv32 = zv.astype(jnp.int32)
    bat32 = x_rv_batch.astype(jnp.int32)
    zeros = jnp.zeros((ACC_ROWS * D,), jnp.float32)
    parts = _sc_pool(out, zv32, bat32, zeros)
    tc_part = _tc_pool(zv32[:NTC].reshape(NB, 1, RB),
                       bat32[:NTC].reshape(NB, 1, RB),
                       out[:NTC])
    summed = _combine(parts, tc_part.reshape(SEG * D))
    return summed.reshape(B, 2 * D)
